# baseline (speedup 1.0000x reference)
import jax
import jax.numpy as jnp
from jax.experimental import pallas as pl


BLOCK_TOKENS = 2048


def _gate_kernel(x_ref, wt_ref, b_ref, out_ref):
    x = x_ref[...]
    wt = wt_ref[...]
    logits = jnp.dot(x, wt, preferred_element_type=jnp.float32) + b_ref[...]
    m = jnp.max(logits, axis=-1, keepdims=True)
    e = jnp.exp(logits - m)
    out_ref[...] = e / jnp.sum(e, axis=-1, keepdims=True)


def kernel(payload_tensor, W, b):
    B, S, D = payload_tensor.shape
    C = W.shape[0]
    T = B * S
    x2 = payload_tensor.reshape(T, D)
    wt = W.T
    b2 = b.reshape(1, C)

    grid = (T // BLOCK_TOKENS,)
    out = pl.pallas_call(
        _gate_kernel,
        grid=grid,
        in_specs=[
            pl.BlockSpec((BLOCK_TOKENS, D), lambda i: (i, 0)),
            pl.BlockSpec((D, C), lambda i: (0, 0)),
            pl.BlockSpec((1, C), lambda i: (0, 0)),
        ],
        out_specs=pl.BlockSpec((BLOCK_TOKENS, C), lambda i: (i, 0)),
        out_shape=jax.ShapeDtypeStruct((T, C), jnp.float32),
    )(x2, wt, b2)
    return out.reshape(B, S, C)
